# baseline (speedup 1.0000x reference)
import functools

import jax
import jax.numpy as jnp
from jax import lax
from jax.experimental import pallas as pl
from jax.experimental.pallas import tpu as pltpu
from jax.experimental.pallas import tpu_sc as plsc

_E = 8
_H = 1024
_I = 2816
_T = 32
_TI = 256
_L = 16


def _sc_router_body(logits_hbm, out_hbm, logits_v, w_v):
    cid = lax.axis_index("c")
    sid = lax.axis_index("s")

    @pl.when(jnp.logical_and(cid == 0, sid == 0))
    def _():
        pltpu.sync_copy(logits_hbm, logits_v)
        for c in range(_T // _L):
            sl = pl.ds(c * _L, _L)
            logit = [logits_v[e, sl] for e in range(_E)]
            m = logit[0]
            for e in range(1, _E):
                m = jnp.maximum(m, logit[e])
            ex = [jnp.exp(logit[e] - m) for e in range(_E)]
            s = ex[0]
            for e in range(1, _E):
                s = s + ex[e]
            prob = [ex[e] / s for e in range(_E)]
            m1 = prob[0]
            i1 = jnp.zeros((_L,), jnp.int32)
            for e in range(1, _E):
                gt = prob[e] > m1
                i1 = jnp.where(gt, e, i1)
                m1 = jnp.where(gt, prob[e], m1)
            m2 = jnp.full((_L,), -1.0, jnp.float32)
            i2 = jnp.zeros((_L,), jnp.int32)
            for e in range(_E):
                gt = jnp.logical_and(i1 != e, prob[e] > m2)
                i2 = jnp.where(gt, e, i2)
                m2 = jnp.where(gt, prob[e], m2)
            denom = m1 + m2
            w1 = m1 / denom
            w2 = m2 / denom
            for e in range(_E):
                w_v[e, sl] = jnp.where(
                    i1 == e, w1, jnp.where(i2 == e, w2, jnp.zeros((_L,), jnp.float32))
                )
        pltpu.sync_copy(w_v, out_hbm)


def _sc_router(logits_et):
    mesh = plsc.VectorSubcoreMesh(core_axis_name="c", subcore_axis_name="s")
    fn = functools.partial(
        pl.kernel,
        mesh=mesh,
        out_type=jax.ShapeDtypeStruct((_E, _T), jnp.float32),
        scratch_types=[
            pltpu.VMEM((_E, _T), jnp.float32),
            pltpu.VMEM((_E, _T), jnp.float32),
        ],
    )(_sc_router_body)
    return fn(logits_et)


_TH = 256


def _tc_act_body(x_ref, wg_ref, wu_ref, act_ref):
    x = x_ref[...]
    gate_w = wg_ref[0, 0]
    up_w = wu_ref[0, 0]
    g = lax.dot_general(x, gate_w, (((1,), (1,)), ((), ())),
                        preferred_element_type=jnp.float32)
    u = lax.dot_general(x, up_w, (((1,), (1,)), ((), ())),
                        preferred_element_type=jnp.float32)
    act_ref[0] = (g * jax.nn.sigmoid(g)) * u


def _tc_down_body(act_ref, w_ref, w2_ref, out_ref):
    e = pl.program_id(1)

    @pl.when(e == 0)
    def _():
        out_ref[...] = jnp.zeros_like(out_ref)

    a = act_ref[0] * w_ref[0, 0, :][:, None]
    out_ref[...] += lax.dot_general(a, w2_ref[0], (((1,), (1,)), ((), ())),
                                    preferred_element_type=jnp.float32)


def _tc_moe(x, w_et3, w13r, w2):
    act = pl.pallas_call(
        _tc_act_body,
        grid=(_E, _I // _TI),
        in_specs=[
            pl.BlockSpec((_T, _H), lambda e, i: (0, 0)),
            pl.BlockSpec((1, 2, _TI, _H), lambda e, i: (e, 0, i, 0)),
        ],
        out_specs=pl.BlockSpec((1, _T, _TI), lambda e, i: (e, 0, i)),
        out_shape=jax.ShapeDtypeStruct((_E, _T, _I), jnp.float32),
        compiler_params=pltpu.CompilerParams(
            dimension_semantics=("parallel", "parallel"),
        ),
    )(x, w13r)
    return pl.pallas_call(
        _tc_down_body,
        grid=(_H // _TH, _E),
        in_specs=[
            pl.BlockSpec((1, _T, _I), lambda h, e: (e, 0, 0)),
            pl.BlockSpec((1, 1, _T), lambda h, e: (e, 0, 0)),
            pl.BlockSpec((1, _TH, _I), lambda h, e: (e, h, 0)),
        ],
        out_specs=pl.BlockSpec((_T, _TH), lambda h, e: (0, h)),
        out_shape=jax.ShapeDtypeStruct((_T, _H), jnp.float32),
        compiler_params=pltpu.CompilerParams(
            dimension_semantics=("arbitrary", "arbitrary"),
        ),
    )(act, w_et3, w2)


_NT = 2
_NS = 4
_TIS = _I // _NS
_THF = _H // _NT


def _tc_fused_body(x_ref, w_ref, wg0_ref, wg1_ref, wu0_ref, wu1_ref, w2_ref,
                   out_ref, acts_ref):
    e = pl.program_id(0)
    t = pl.program_id(1)

    @pl.when(jnp.logical_and(e == 0, t == 0))
    def _():
        out_ref[...] = jnp.zeros_like(out_ref)

    @pl.when(e < _E)
    def _():
        x = x_ref[...]
        for k, (wg, wu) in enumerate(((wg0_ref, wu0_ref), (wg1_ref, wu1_ref))):
            g = lax.dot_general(x, wg[0, 0], (((1,), (1,)), ((), ())),
                                preferred_element_type=jnp.float32)
            u = lax.dot_general(x, wu[0, 0], (((1,), (1,)), ((), ())),
                                preferred_element_type=jnp.float32)
            a = (g * jax.nn.sigmoid(g)) * u
            acts_ref[pl.ds(e % 2, 1), pl.ds(2 * t + k, 1)] = a[None, None]

    @pl.when(e >= 1)
    def _():
        wrow = w_ref[0, 0, :][:, None]
        acc = jnp.zeros((_T, _THF), jnp.float32)
        for j in range(_NS):
            a = acts_ref[pl.ds((e + 1) % 2, 1), pl.ds(j, 1)][0, 0] * wrow
            acc += lax.dot_general(a, w2_ref[0, :, j], (((1,), (1,)), ((), ())),
                                   preferred_element_type=jnp.float32)
        out_ref[:, pl.ds(t * _THF, _THF)] += acc


def _tc_moe_fused(x, w_et3, w13r, w2m):
    last = _E - 1
    return pl.pallas_call(
        _tc_fused_body,
        grid=(_E + 1, _NT),
        in_specs=[
            pl.BlockSpec((_T, _H), lambda e, t: (0, 0)),
            pl.BlockSpec((1, 1, _T), lambda e, t: (jnp.maximum(e - 1, 0), 0, 0)),
            pl.BlockSpec((1, 1, _TIS, _H), lambda e, t: (
                jnp.minimum(e, last), 0, jnp.where(e == _E, 2, 2 * t), 0)),
            pl.BlockSpec((1, 1, _TIS, _H), lambda e, t: (
                jnp.minimum(e, last), 0, jnp.where(e == _E, 3, 2 * t + 1), 0)),
            pl.BlockSpec((1, 1, _TIS, _H), lambda e, t: (
                jnp.minimum(e, last), 1, jnp.where(e == _E, 2, 2 * t), 0)),
            pl.BlockSpec((1, 1, _TIS, _H), lambda e, t: (
                jnp.minimum(e, last), 1, jnp.where(e == _E, 3, 2 * t + 1), 0)),
            pl.BlockSpec((1, _THF, _NS, _TIS), lambda e, t: (
                jnp.maximum(e - 1, 0), jnp.where(e == 0, 0, t), 0, 0)),
        ],
        out_specs=pl.BlockSpec((_T, _H), lambda e, t: (0, 0)),
        out_shape=jax.ShapeDtypeStruct((_T, _H), jnp.float32),
        scratch_shapes=[pltpu.VMEM((2, _NS, _T, _TIS), jnp.float32)],
        compiler_params=pltpu.CompilerParams(
            dimension_semantics=("arbitrary", "arbitrary"),
        ),
    )(x, w_et3, w13r, w13r, w13r, w13r, w2m)


def kernel(x, router_logits, w13, w2):
    logits_et = router_logits.T
    w_et = _sc_router(logits_et)
    w_et3 = w_et.reshape(_E, 1, _T)
    w13r = w13.reshape(_E, 2, _I, _H)
    w2m = w2.reshape(_E, _H, _NS, _TIS)
    return _tc_moe_fused(x, w_et3, w13r, w2m)


# speedup vs baseline: 2.5917x; 2.5917x over previous
import functools

import jax
import jax.numpy as jnp
from jax import lax
from jax.experimental import pallas as pl
from jax.experimental.pallas import tpu as pltpu
from jax.experimental.pallas import tpu_sc as plsc

_E = 8
_H = 1024
_I = 2816
_T = 32
_TI = 256
_L = 16


def _sc_router_body(logits_hbm, out_hbm, logits_v, w_v):
    cid = lax.axis_index("c")
    sid = lax.axis_index("s")

    @pl.when(jnp.logical_and(cid == 0, sid == 0))
    def _():
        pltpu.sync_copy(logits_hbm, logits_v)
        for c in range(_T // _L):
            sl = pl.ds(c * _L, _L)
            logit = [logits_v[e, sl] for e in range(_E)]
            m = logit[0]
            for e in range(1, _E):
                m = jnp.maximum(m, logit[e])
            ex = [jnp.exp(logit[e] - m) for e in range(_E)]
            s = ex[0]
            for e in range(1, _E):
                s = s + ex[e]
            prob = [ex[e] / s for e in range(_E)]
            m1 = prob[0]
            i1 = jnp.zeros((_L,), jnp.int32)
            for e in range(1, _E):
                gt = prob[e] > m1
                i1 = jnp.where(gt, e, i1)
                m1 = jnp.where(gt, prob[e], m1)
            m2 = jnp.full((_L,), -1.0, jnp.float32)
            i2 = jnp.zeros((_L,), jnp.int32)
            for e in range(_E):
                gt = jnp.logical_and(i1 != e, prob[e] > m2)
                i2 = jnp.where(gt, e, i2)
                m2 = jnp.where(gt, prob[e], m2)
            denom = m1 + m2
            w1 = m1 / denom
            w2 = m2 / denom
            for e in range(_E):
                w_v[e, sl] = jnp.where(
                    i1 == e, w1, jnp.where(i2 == e, w2, jnp.zeros((_L,), jnp.float32))
                )
        pltpu.sync_copy(w_v, out_hbm)


def _sc_router(logits_et):
    mesh = plsc.VectorSubcoreMesh(core_axis_name="c", subcore_axis_name="s")
    fn = functools.partial(
        pl.kernel,
        mesh=mesh,
        out_type=jax.ShapeDtypeStruct((_E, _T), jnp.float32),
        scratch_types=[
            pltpu.VMEM((_E, _T), jnp.float32),
            pltpu.VMEM((_E, _T), jnp.float32),
        ],
    )(_sc_router_body)
    return fn(logits_et)


_TH = 256


def _tc_act_body(x_ref, wg_ref, wu_ref, act_ref):
    x = x_ref[...]
    gate_w = wg_ref[0, 0]
    up_w = wu_ref[0, 0]
    g = lax.dot_general(x, gate_w, (((1,), (1,)), ((), ())),
                        preferred_element_type=jnp.float32)
    u = lax.dot_general(x, up_w, (((1,), (1,)), ((), ())),
                        preferred_element_type=jnp.float32)
    act_ref[0] = (g * jax.nn.sigmoid(g)) * u


def _tc_down_body(act_ref, w_ref, w2_ref, out_ref):
    e = pl.program_id(1)

    @pl.when(e == 0)
    def _():
        out_ref[...] = jnp.zeros_like(out_ref)

    a = act_ref[0] * w_ref[0, 0, :][:, None]
    out_ref[...] += lax.dot_general(a, w2_ref[0], (((1,), (1,)), ((), ())),
                                    preferred_element_type=jnp.float32)


def _tc_moe(x, w_et3, w13r, w2):
    act = pl.pallas_call(
        _tc_act_body,
        grid=(_E, _I // _TI),
        in_specs=[
            pl.BlockSpec((_T, _H), lambda e, i: (0, 0)),
            pl.BlockSpec((1, 2, _TI, _H), lambda e, i: (e, 0, i, 0)),
        ],
        out_specs=pl.BlockSpec((1, _T, _TI), lambda e, i: (e, 0, i)),
        out_shape=jax.ShapeDtypeStruct((_E, _T, _I), jnp.float32),
        compiler_params=pltpu.CompilerParams(
            dimension_semantics=("parallel", "parallel"),
        ),
    )(x, w13r)
    return pl.pallas_call(
        _tc_down_body,
        grid=(_H // _TH, _E),
        in_specs=[
            pl.BlockSpec((1, _T, _I), lambda h, e: (e, 0, 0)),
            pl.BlockSpec((1, 1, _T), lambda h, e: (e, 0, 0)),
            pl.BlockSpec((1, _TH, _I), lambda h, e: (e, h, 0)),
        ],
        out_specs=pl.BlockSpec((_T, _TH), lambda h, e: (0, h)),
        out_shape=jax.ShapeDtypeStruct((_T, _H), jnp.float32),
        compiler_params=pltpu.CompilerParams(
            dimension_semantics=("arbitrary", "arbitrary"),
        ),
    )(act, w_et3, w2)


_NT = 2
_TIF = _I // _NT
_THF = _H // _NT


def _tc_fused_body(x_ref, w_ref, wg_ref, wu_ref, w2_ref, out_ref, acts_ref):
    e = pl.program_id(0)
    t = pl.program_id(1)

    @pl.when(jnp.logical_and(e == 0, t == 0))
    def _():
        out_ref[...] = jnp.zeros_like(out_ref)

    @pl.when(e < _E)
    def _():
        x = x_ref[...]
        g = lax.dot_general(x, wg_ref[0, 0], (((1,), (1,)), ((), ())),
                            preferred_element_type=jnp.float32)
        u = lax.dot_general(x, wu_ref[0, 0], (((1,), (1,)), ((), ())),
                            preferred_element_type=jnp.float32)
        a = (g * jax.nn.sigmoid(g)) * u
        acts_ref[pl.ds(e % 2, 1), :, pl.ds(t * _TIF, _TIF)] = a[None]

    @pl.when(e >= 1)
    def _():
        prev = acts_ref[pl.ds((e + 1) % 2, 1)][0]
        a = prev * w_ref[0, 0, :][:, None]
        out_ref[:, pl.ds(t * _THF, _THF)] += lax.dot_general(
            a, w2_ref[0], (((1,), (1,)), ((), ())),
            preferred_element_type=jnp.float32)


def _tc_moe_fused(x, w_et3, w13r, w2):
    last = _E - 1
    return pl.pallas_call(
        _tc_fused_body,
        grid=(_E + 1, _NT),
        in_specs=[
            pl.BlockSpec((_T, _H), lambda e, t: (0, 0)),
            pl.BlockSpec((1, 1, _T), lambda e, t: (jnp.maximum(e - 1, 0), 0, 0)),
            pl.BlockSpec((1, 1, _TIF, _H), lambda e, t: (
                jnp.minimum(e, last), 0, jnp.where(e == _E, _NT - 1, t), 0)),
            pl.BlockSpec((1, 1, _TIF, _H), lambda e, t: (
                jnp.minimum(e, last), 1, jnp.where(e == _E, _NT - 1, t), 0)),
            pl.BlockSpec((1, _THF, _I), lambda e, t: (
                jnp.maximum(e - 1, 0), jnp.where(e == 0, 0, t), 0)),
        ],
        out_specs=pl.BlockSpec((_T, _H), lambda e, t: (0, 0)),
        out_shape=jax.ShapeDtypeStruct((_T, _H), jnp.float32),
        scratch_shapes=[pltpu.VMEM((2, _T, _I), jnp.float32)],
        compiler_params=pltpu.CompilerParams(
            dimension_semantics=("arbitrary", "arbitrary"),
        ),
    )(x, w_et3, w13r, w13r, w2)


def kernel(x, router_logits, w13, w2):
    logits_et = router_logits.T
    w_et = _sc_router(logits_et)
    w_et3 = w_et.reshape(_E, 1, _T)
    w13r = w13.reshape(_E, 2, _I, _H)
    return _tc_moe_fused(x, w_et3, w13r, w2)
